# baseline (speedup 1.0000x reference)
import functools
import math

import jax
import jax.numpy as jnp
from jax import lax
from jax.experimental import pallas as pl
from jax.experimental.pallas import tpu as pltpu
from jax.experimental.pallas import tpu_sc as plsc

_GRID = 4
_STRIDE = 2
_G = 128


def _sampled_idx(key, size, grid, m):
    nblocks = math.ceil(size / grid)
    keys = jax.random.split(key, nblocks)
    perms = jax.vmap(lambda k: jax.random.permutation(k, grid)[:m])(keys)
    idx = (perms + jnp.arange(nblocks)[:, None] * grid).reshape(-1)
    return jnp.clip(jnp.sort(idx), 0, size - 1)


@functools.cache
def _make_sc_gather(R, W, NO):
    info = plsc.get_sparse_core_info()
    nc, ns = info.num_cores, info.num_subcores
    nw = nc * ns
    rows_per = R // nw
    nchunks = rows_per // _G
    assert rows_per % _G == 0 and R % nw == 0 and NO % 16 == 0
    mesh = plsc.VectorSubcoreMesh(core_axis_name="c", subcore_axis_name="s")

    @functools.partial(
        pl.kernel,
        mesh=mesh,
        compiler_params=pltpu.CompilerParams(
            needs_layout_passes=False, use_tc_tiling_on_sc=False),
        out_type=jax.ShapeDtypeStruct((R, NO), jnp.float32),
        scratch_types=[
            pltpu.VMEM((_G,), jnp.int32),
            pltpu.VMEM((NO,), jnp.int32),
            pltpu.VMEM((_G, W), jnp.float32),
            pltpu.VMEM((_G, NO), jnp.float32),
            pltpu.SemaphoreType.DMA,
        ],
    )
    def body(x2, ridx, cq, out, idx_v, cq_v, rows_v, out_v, sem):
        wid = lax.axis_index("s") * nc + lax.axis_index("c")
        base0 = wid * rows_per
        pltpu.sync_copy(cq, cq_v)

        def chunk(t, carry):
            base = pl.multiple_of(base0 + t * _G, _G)
            pltpu.sync_copy(ridx.at[pl.ds(base, _G)], idx_v)
            pltpu.async_copy(x2.at[idx_v], rows_v, sem).wait()

            def row(g, c2):
                gs = jnp.full((16,), g, jnp.int32)
                for j in range(NO // 16):
                    cvec = cq_v[pl.ds(j * 16, 16)]
                    out_v[g, pl.ds(j * 16, 16)] = plsc.load_gather(
                        rows_v, [gs, cvec])
                return c2

            lax.fori_loop(0, _G, row, 0)
            pltpu.sync_copy(out_v, out.at[pl.ds(base, _G)])
            return carry

        lax.fori_loop(0, nchunks, chunk, 0)

    return body


def kernel(x):
    B, C, H, W = x.shape
    m = _GRID // _STRIDE
    kr, kc = jax.random.split(jax.random.key(42))
    r_p = _sampled_idx(kr, H, _GRID, m)
    c_q = _sampled_idx(kc, W, _GRID, m)
    nr, no = r_p.shape[0], c_q.shape[0]
    planes = B * C
    ridx = (jnp.arange(planes, dtype=jnp.int32)[:, None] * H
            + r_p[None, :].astype(jnp.int32)).reshape(-1)
    x2 = x.reshape(planes * H, W)
    out = _make_sc_gather(planes * nr, W, no)(x2, ridx, c_q.astype(jnp.int32))
    return out.reshape(B, C, nr, no)


# speedup vs baseline: 1.7570x; 1.7570x over previous
import functools
import math

import jax
import jax.numpy as jnp
from jax import lax
from jax.experimental import pallas as pl
from jax.experimental.pallas import tpu as pltpu
from jax.experimental.pallas import tpu_sc as plsc

_GRID = 4
_STRIDE = 2
_G = 96


def _sampled_idx(key, size, grid, m):
    nblocks = math.ceil(size / grid)
    keys = jax.random.split(key, nblocks)
    perms = jax.vmap(lambda k: jax.random.permutation(k, grid)[:m])(keys)
    idx = (perms + jnp.arange(nblocks)[:, None] * grid).reshape(-1)
    return jnp.clip(jnp.sort(idx), 0, size - 1)


@functools.cache
def _make_sc_gather(R, W, NO):
    info = plsc.get_sparse_core_info()
    nc, ns = info.num_cores, info.num_subcores
    nw = nc * ns
    rows_per = R // nw
    nchunks = rows_per // _G
    assert rows_per % _G == 0 and R % nw == 0 and NO % 16 == 0
    assert nchunks % 2 == 0
    mesh = plsc.VectorSubcoreMesh(core_axis_name="c", subcore_axis_name="s")

    @functools.partial(
        pl.kernel,
        mesh=mesh,
        compiler_params=pltpu.CompilerParams(
            needs_layout_passes=False, use_tc_tiling_on_sc=False),
        out_type=jax.ShapeDtypeStruct((R, NO), jnp.float32),
        scratch_types=[
            pltpu.VMEM((_G,), jnp.int32),
            pltpu.VMEM((_G,), jnp.int32),
            pltpu.VMEM((NO,), jnp.int32),
            pltpu.VMEM((_G, W), jnp.float32),
            pltpu.VMEM((_G, W), jnp.float32),
            pltpu.VMEM((_G, NO), jnp.float32),
            pltpu.VMEM((_G, NO), jnp.float32),
            pltpu.SemaphoreType.DMA,
            pltpu.SemaphoreType.DMA,
            pltpu.SemaphoreType.DMA,
            pltpu.SemaphoreType.DMA,
        ],
    )
    def body(x2, ridx, cq, out, idx_a, idx_b, cq_v, rows_a, rows_b,
             out_a, out_b, isem_a, isem_b, osem_a, osem_b):
        wid = lax.axis_index("s") * nc + lax.axis_index("c")
        base0 = wid * rows_per
        idxs = (idx_a, idx_b)
        rows = (rows_a, rows_b)
        outs = (out_a, out_b)
        isems = (isem_a, isem_b)
        osems = (osem_a, osem_b)
        pltpu.sync_copy(cq, cq_v)
        cqv = [cq_v[pl.ds(16 * j, 16)] for j in range(NO // 16)]

        def chunk_base(t):
            return pl.multiple_of(base0 + t * _G, _G)

        def start_in(t, b):
            pltpu.sync_copy(ridx.at[pl.ds(chunk_base(t), _G)], idxs[b])
            pltpu.async_copy(x2.at[idxs[b]], rows[b], isems[b])

        start_in(0, 0)

        def pair(tt, carry):
            for b in range(2):
                t = tt * 2 + b
                pltpu.make_async_copy(x2.at[idxs[b]], rows[b], isems[b]).wait()

                @pl.when(t + 1 < nchunks)
                def _():
                    start_in(t + 1, 1 - b)

                @pl.when(t >= 2)
                def _():
                    pltpu.make_async_copy(
                        outs[b], out.at[pl.ds(chunk_base(t - 2), _G)],
                        osems[b]).wait()

                @plsc.parallel_loop(0, _G, 1, unroll=2)
                def _row(g):
                    gs = jnp.full((16,), g, jnp.int32)
                    for j in range(NO // 16):
                        outs[b][g, pl.ds(16 * j, 16)] = plsc.load_gather(
                            rows[b], [gs, cqv[j]])

                pltpu.async_copy(
                    outs[b], out.at[pl.ds(chunk_base(t), _G)], osems[b])
            return carry

        lax.fori_loop(0, nchunks // 2, pair, 0)
        for b in range(2):
            t = nchunks - 2 + b
            pltpu.make_async_copy(
                outs[b], out.at[pl.ds(chunk_base(t), _G)], osems[b]).wait()

    return body


def kernel(x):
    B, C, H, W = x.shape
    m = _GRID // _STRIDE
    kr, kc = jax.random.split(jax.random.key(42))
    r_p = _sampled_idx(kr, H, _GRID, m)
    c_q = _sampled_idx(kc, W, _GRID, m)
    nr, no = r_p.shape[0], c_q.shape[0]
    planes = B * C
    ridx = (jnp.arange(planes, dtype=jnp.int32)[:, None] * H
            + r_p[None, :].astype(jnp.int32)).reshape(-1)
    x2 = x.reshape(planes * H, W)
    out = _make_sc_gather(planes * nr, W, no)(x2, ridx, c_q.astype(jnp.int32))
    return out.reshape(B, C, nr, no)


# speedup vs baseline: 4.1721x; 2.3746x over previous
import functools
import math

import jax
import jax.numpy as jnp
import numpy as np
from jax import lax
from jax.experimental import pallas as pl
from jax.experimental.pallas import tpu as pltpu
from jax.experimental.pallas import tpu_sc as plsc

_GRID = 4
_STRIDE = 2
_G = 96
_LW = 128


def _sampled_idx(key, size, grid, m):
    nblocks = math.ceil(size / grid)
    keys = jax.random.split(key, nblocks)
    perms = jax.vmap(lambda k: jax.random.permutation(k, grid)[:m])(keys)
    idx = (perms + jnp.arange(nblocks)[:, None] * grid).reshape(-1)
    return jnp.clip(jnp.sort(idx), 0, size - 1)


_RP384 = np.array([
    2,3,5,6,8,9,13,14,17,18,22,23,24,25,30,31,33,34,38,39,41,42,45,47,48,51,
    53,54,56,57,60,61,65,66,68,70,73,75,77,79,80,81,85,87,89,91,94,95,96,99,
    101,103,104,105,110,111,112,114,116,117,121,123,125,127,128,129,133,135,
    137,139,140,141,144,146,148,150,152,154,156,157,161,163,164,166,169,171,
    174,175,178,179,180,181,185,186,189,190,194,195,196,198,200,203,204,206,
    209,211,214,215,216,218,221,223,224,227,229,230,232,233,237,239,241,242,
    244,246,250,251,254,255,256,257,262,263,266,267,268,271,273,274,276,279,
    282,283,284,287,289,291,293,294,296,298,300,303,304,307,308,310,313,315,
    317,318,320,323,325,326,330,331,332,334,338,339,341,342,344,346,349,350,
    353,355,357,358,362,363,366,367,368,369,373,374,377,379,381,382],
    dtype=np.int64)
_CQ384 = np.array([
    1,3,5,6,8,9,13,14,17,18,21,23,24,25,29,30,32,34,36,39,42,43,45,47,48,49,
    52,54,56,59,60,63,64,67,69,70,74,75,76,77,81,82,85,86,90,91,93,94,96,97,
    100,102,104,106,108,111,113,114,116,118,120,121,126,127,130,131,132,135,
    136,139,140,143,146,147,149,151,152,155,156,157,160,163,164,165,168,169,
    173,174,178,179,180,183,184,186,188,189,193,195,196,199,200,203,204,207,
    209,211,212,213,217,219,221,222,224,225,228,229,232,233,236,239,240,241,
    245,247,248,251,253,255,256,257,262,263,264,266,269,271,274,275,277,278,
    281,283,286,287,288,289,293,295,296,298,300,301,304,306,308,309,312,313,
    316,317,320,321,325,326,330,331,333,335,337,339,342,343,344,347,348,350,
    352,353,357,359,361,362,364,367,369,371,372,375,376,377,381,383],
    dtype=np.int64)


@functools.cache
def _make_sc_gather(nseg_rows, NR, NO, nwb):
    info = plsc.get_sparse_core_info()
    nc, ns = info.num_cores, info.num_subcores
    nw = nc * ns
    R = nseg_rows
    rows_per = R // nw
    nchunks = rows_per // _G
    assert rows_per % _G == 0 and R % nw == 0 and NO % 16 == 0
    assert nchunks % 2 == 0
    mesh = plsc.VectorSubcoreMesh(core_axis_name="c", subcore_axis_name="s")

    @functools.partial(
        pl.kernel,
        mesh=mesh,
        compiler_params=pltpu.CompilerParams(
            needs_layout_passes=False, use_tc_tiling_on_sc=False),
        out_type=jax.ShapeDtypeStruct((R, NO), jnp.float32),
        scratch_types=[
            pltpu.VMEM((nwb, _G), jnp.int32),
            pltpu.VMEM((nwb, _G), jnp.int32),
            pltpu.VMEM((NO,), jnp.int32),
            pltpu.VMEM((nwb, _G, _LW), jnp.float32),
            pltpu.VMEM((nwb, _G, _LW), jnp.float32),
            pltpu.VMEM((_G, NO), jnp.float32),
            pltpu.VMEM((_G, NO), jnp.float32),
            pltpu.SemaphoreType.DMA,
            pltpu.SemaphoreType.DMA,
            pltpu.SemaphoreType.DMA,
            pltpu.SemaphoreType.DMA,
        ],
    )
    def body(seg, sidx, cqm, out, idx_a, idx_b, cq_v, rows_a, rows_b,
             out_a, out_b, isem_a, isem_b, osem_a, osem_b):
        wid = lax.axis_index("s") * nc + lax.axis_index("c")
        chunk0 = wid * nchunks
        idxs = (idx_a, idx_b)
        rows = (rows_a, rows_b)
        outs = (out_a, out_b)
        isems = (isem_a, isem_b)
        osems = (osem_a, osem_b)
        pltpu.sync_copy(cqm, cq_v)
        cqv = [cq_v[pl.ds(16 * j, 16)] for j in range(NO // 16)]

        def start_in(t, b):
            pltpu.sync_copy(sidx.at[chunk0 + t], idxs[b])
            for k in range(nwb):
                pltpu.async_copy(seg.at[idxs[b].at[k]], rows[b].at[k],
                                 isems[b])

        def wait_in(b):
            for k in range(nwb):
                pltpu.make_async_copy(seg.at[idxs[b].at[k]], rows[b].at[k],
                                      isems[b]).wait()

        def out_base(t):
            return pl.multiple_of((chunk0 + t) * _G, _G)

        start_in(0, 0)

        def pair(tt, carry):
            for b in range(2):
                t = tt * 2 + b
                wait_in(b)

                @pl.when(t + 1 < nchunks)
                def _():
                    start_in(t + 1, 1 - b)

                @pl.when(t >= 2)
                def _():
                    pltpu.make_async_copy(
                        outs[b], out.at[pl.ds(out_base(t - 2), _G)],
                        osems[b]).wait()

                @plsc.parallel_loop(0, _G, 1, unroll=2)
                def _row(g):
                    gs = jnp.full((16,), g, jnp.int32)
                    for j in range(NO // 16):
                        ks = jnp.full((16,), j // 4, jnp.int32)
                        outs[b][g, pl.ds(16 * j, 16)] = plsc.load_gather(
                            rows[b], [ks, gs, cqv[j]])

                pltpu.async_copy(
                    outs[b], out.at[pl.ds(out_base(t), _G)], osems[b])
            return carry

        lax.fori_loop(0, nchunks // 2, pair, 0)
        for b in range(2):
            t = nchunks - 2 + b
            pltpu.make_async_copy(
                outs[b], out.at[pl.ds(out_base(t), _G)], osems[b]).wait()

    return body


def kernel(x):
    B, C, H, W = x.shape
    assert (H, W) == (384, 384)
    r_p, c_q = _RP384, _CQ384
    nr, no = r_p.shape[0], c_q.shape[0]
    planes = B * C
    nwb = W // _LW
    nh = H // 8
    R = planes * nr
    nchunks_total = R // _G

    seg = (x.reshape(B, C, nh, 8, nwb, _LW)
           .transpose(0, 1, 2, 4, 3, 5)
           .reshape(planes * nh * nwb * 8, _LW))

    r = np.arange(R, dtype=np.int64)
    p, i = r // nr, r % nr
    h = r_p[i]
    sid = (((p * nh + h // 8) * nwb)[None, :]
           + np.arange(nwb, dtype=np.int64)[:, None]) * 8 + (h % 8)[None, :]
    sid = (sid.reshape(nwb, nchunks_total, _G).transpose(1, 0, 2)
           .astype(np.int32))

    assert np.all(c_q // _LW == np.arange(no) // (no // nwb))
    cqm = (c_q % _LW).astype(np.int32)

    out = _make_sc_gather(R, nr, no, nwb)(
        seg, jnp.asarray(sid), jnp.asarray(cqm))
    return out.reshape(B, C, nr, no)


# speedup vs baseline: 7.1510x; 1.7140x over previous
import functools

import jax
import jax.numpy as jnp
import numpy as np
from jax import lax
from jax.experimental import pallas as pl
from jax.experimental.pallas import tpu as pltpu
from jax.experimental.pallas import tpu_sc as plsc

_G = 96
_LW = 128

_RP384 = np.array([
    2,3,5,6,8,9,13,14,17,18,22,23,24,25,30,31,33,34,38,39,41,42,45,47,48,51,
    53,54,56,57,60,61,65,66,68,70,73,75,77,79,80,81,85,87,89,91,94,95,96,99,
    101,103,104,105,110,111,112,114,116,117,121,123,125,127,128,129,133,135,
    137,139,140,141,144,146,148,150,152,154,156,157,161,163,164,166,169,171,
    174,175,178,179,180,181,185,186,189,190,194,195,196,198,200,203,204,206,
    209,211,214,215,216,218,221,223,224,227,229,230,232,233,237,239,241,242,
    244,246,250,251,254,255,256,257,262,263,266,267,268,271,273,274,276,279,
    282,283,284,287,289,291,293,294,296,298,300,303,304,307,308,310,313,315,
    317,318,320,323,325,326,330,331,332,334,338,339,341,342,344,346,349,350,
    353,355,357,358,362,363,366,367,368,369,373,374,377,379,381,382],
    dtype=np.int64)
_CQ384 = np.array([
    1,3,5,6,8,9,13,14,17,18,21,23,24,25,29,30,32,34,36,39,42,43,45,47,48,49,
    52,54,56,59,60,63,64,67,69,70,74,75,76,77,81,82,85,86,90,91,93,94,96,97,
    100,102,104,106,108,111,113,114,116,118,120,121,126,127,130,131,132,135,
    136,139,140,143,146,147,149,151,152,155,156,157,160,163,164,165,168,169,
    173,174,178,179,180,183,184,186,188,189,193,195,196,199,200,203,204,207,
    209,211,212,213,217,219,221,222,224,225,228,229,232,233,236,239,240,241,
    245,247,248,251,253,255,256,257,262,263,264,266,269,271,274,275,277,278,
    281,283,286,287,288,289,293,295,296,298,300,301,304,306,308,309,312,313,
    316,317,320,321,325,326,330,331,333,335,337,339,342,343,344,347,348,350,
    352,353,357,359,361,362,364,367,369,371,372,375,376,377,381,383],
    dtype=np.int64)


@functools.cache
def _make_sc_gather(R, S, NO, nwb, nob):
    info = plsc.get_sparse_core_info()
    nc, ns = info.num_cores, info.num_subcores
    nw = nc * ns
    rows_per = R // nw
    nchunks = rows_per // _G
    assert rows_per % _G == 0 and R % nw == 0 and NO % 16 == 0
    assert nchunks % 2 == 0
    mesh = plsc.VectorSubcoreMesh(core_axis_name="c", subcore_axis_name="s")

    @functools.partial(
        pl.kernel,
        mesh=mesh,
        compiler_params=pltpu.CompilerParams(
            needs_layout_passes=False, use_tc_tiling_on_sc=False),
        out_type=jax.ShapeDtypeStruct((S, _LW), jnp.float32),
        scratch_types=[
            pltpu.VMEM((nwb, _G), jnp.int32),
            pltpu.VMEM((nwb, _G), jnp.int32),
            pltpu.VMEM((nob, _G), jnp.int32),
            pltpu.VMEM((nob, _G), jnp.int32),
            pltpu.VMEM((NO,), jnp.int32),
            pltpu.VMEM((nwb, _G, _LW), jnp.float32),
            pltpu.VMEM((nwb, _G, _LW), jnp.float32),
            pltpu.VMEM((nob, _G, _LW), jnp.float32),
            pltpu.VMEM((nob, _G, _LW), jnp.float32),
            pltpu.SemaphoreType.DMA,
            pltpu.SemaphoreType.DMA,
            pltpu.SemaphoreType.DMA,
            pltpu.SemaphoreType.DMA,
        ],
    )
    def body(seg, sidx, soidx, cqm, out, idx_a, idx_b, oidx_a, oidx_b, cq_v,
             rows_a, rows_b, oseg_a, oseg_b, isem_a, isem_b, osem_a, osem_b):
        wid = lax.axis_index("s") * nc + lax.axis_index("c")
        chunk0 = wid * nchunks
        idxs = (idx_a, idx_b)
        oidxs = (oidx_a, oidx_b)
        rows = (rows_a, rows_b)
        osegs = (oseg_a, oseg_b)
        isems = (isem_a, isem_b)
        osems = (osem_a, osem_b)
        pltpu.sync_copy(cqm, cq_v)
        cqv = [cq_v[pl.ds(16 * j, 16)] for j in range(NO // 16)]

        def start_in(t, b):
            pltpu.sync_copy(sidx.at[chunk0 + t], idxs[b])
            for k in range(nwb):
                pltpu.async_copy(seg.at[idxs[b].at[k]], rows[b].at[k],
                                 isems[b])

        def wait_in(b):
            for k in range(nwb):
                pltpu.make_async_copy(seg.at[idxs[b].at[k]], rows[b].at[k],
                                      isems[b]).wait()

        def start_out(b):
            for k in range(nob):
                pltpu.async_copy(osegs[b].at[k], out.at[oidxs[b].at[k]],
                                 osems[b])

        def wait_out(b):
            for k in range(nob):
                pltpu.make_async_copy(osegs[b].at[k], out.at[oidxs[b].at[k]],
                                      osems[b]).wait()

        start_in(0, 0)

        def pair(tt, carry):
            for b in range(2):
                t = tt * 2 + b
                wait_in(b)

                @pl.when(t + 1 < nchunks)
                def _():
                    start_in(t + 1, 1 - b)

                @pl.when(t >= 2)
                def _():
                    wait_out(b)

                pltpu.sync_copy(soidx.at[chunk0 + t], oidxs[b])

                @plsc.parallel_loop(0, _G, 1, unroll=2)
                def _row(g):
                    gs = jnp.full((16,), g, jnp.int32)
                    for j in range(NO // 16):
                        ks = jnp.full((16,), j // 4, jnp.int32)
                        v = plsc.load_gather(rows[b], [ks, gs, cqv[j]])
                        osegs[b][j // 8, g, pl.ds(16 * (j % 8), 16)] = v

                start_out(b)
            return carry

        lax.fori_loop(0, nchunks // 2, pair, 0)
        for b in range(2):
            wait_out(b)

    return body


def kernel(x):
    B, C, H, W = x.shape
    assert (H, W) == (384, 384)
    r_p, c_q = _RP384, _CQ384
    nr, no = r_p.shape[0], c_q.shape[0]
    planes = B * C
    nwb = W // _LW
    nob = (no + _LW - 1) // _LW
    nh = H // 8
    noh = nr // 8
    R = planes * nr
    S = planes * noh * nob * 8
    nchunks_total = R // _G

    seg = (x.reshape(B, C, nh, 8, nwb, _LW)
           .transpose(0, 1, 2, 4, 3, 5)
           .reshape(planes * nh * nwb * 8, _LW))

    r = np.arange(R, dtype=np.int64)
    p, i = r // nr, r % nr
    h = r_p[i]
    sid = (((p * nh + h // 8) * nwb)[None, :]
           + np.arange(nwb, dtype=np.int64)[:, None]) * 8 + (h % 8)[None, :]
    sid = (sid.reshape(nwb, nchunks_total, _G).transpose(1, 0, 2)
           .astype(np.int32))

    soid = (((p * noh + i // 8) * nob)[None, :]
            + np.arange(nob, dtype=np.int64)[:, None]) * 8 + (i % 8)[None, :]
    soid = (soid.reshape(nob, nchunks_total, _G).transpose(1, 0, 2)
            .astype(np.int32))

    assert np.all(c_q // _LW == np.arange(no) // (no // nwb))
    cqm = (c_q % _LW).astype(np.int32)

    y = _make_sc_gather(R, S, no, nwb, nob)(
        seg, jnp.asarray(sid), jnp.asarray(soid), jnp.asarray(cqm))
    y = (y.reshape(B, C, noh, nob, 8, _LW)
         .transpose(0, 1, 2, 4, 3, 5)
         .reshape(B, C, nr, nob * _LW))
    return y[:, :, :, :no]
